# Initial kernel scaffold; baseline (speedup 1.0000x reference)
import jax
import jax.numpy as jnp
from jax.experimental import pallas as pl


def kernel(xyz, points, W_0_0, b_0_0, W_0_1, b_0_1, W_0_2, b_0_2, W_1_0, b_1_0, W_1_1, b_1_1, W_1_2, b_1_2):
    raise NotImplementedError("write your pallas kernel here")


# speedup vs baseline: 1.7264x; 1.7264x over previous
import functools
import jax
import jax.numpy as jnp
from jax.experimental import pallas as pl
from jax.experimental.pallas import tpu as pltpu

B = 4
N = 8192
S = 512
CIN = 32
CT = 40
RAD = (0.2, 0.4)
KS = (32, 64)
S_BLK = 128


def _fps_body(xyz_ref, out_ref):
    x = xyz_ref[:, 0, :]
    y = xyz_ref[:, 1, :]
    z = xyz_ref[:, 2, :]
    lane = jax.lax.broadcasted_iota(jnp.int32, (B, N), 1)

    def step(t, carry):
        distance, farthest = carry
        sel = (lane == farthest).astype(jnp.float32)
        cx = jnp.sum(x * sel, axis=-1, keepdims=True)
        cy = jnp.sum(y * sel, axis=-1, keepdims=True)
        cz = jnp.sum(z * sel, axis=-1, keepdims=True)
        cvec = jnp.concatenate([cx, cy, cz], axis=-1)
        out_ref[:, pl.ds(t, 1), :] = cvec[:, None, :]
        dx = x - cx
        dy = y - cy
        dz = z - cz
        dist = dx * dx + dy * dy + dz * dz
        distance = jnp.minimum(distance, dist)
        m = jnp.max(distance, axis=-1, keepdims=True)
        new_far = jnp.min(jnp.where(distance == m, lane, N), axis=-1,
                          keepdims=True)
        return distance, new_far

    dist0 = jnp.full((B, N), 1e10, dtype=jnp.float32)
    far0 = jnp.zeros((B, 1), dtype=jnp.int32)
    jax.lax.fori_loop(0, S, step, (dist0, far0))


def _fps(xyz):
    return pl.pallas_call(
        _fps_body,
        out_shape=jax.ShapeDtypeStruct((B, S, 3), jnp.float32),
    )(xyz)


def _ball_query_gather(xyz_t, table, new_xyz, radius, k):
    sqr = (jnp.sum(new_xyz ** 2, -1)[:, :, None]
           + jnp.sum(xyz_t ** 2, -1)[:, None, :]
           - 2.0 * jnp.einsum('bsc,bnc->bsn', new_xyz, xyz_t))
    gidx = jnp.broadcast_to(jnp.arange(N, dtype=jnp.int32), (B, S, N))
    gidx = jnp.where(sqr > radius ** 2, N, gidx)
    gidx = jnp.sort(gidx, axis=-1)[:, :, :k]
    first = gidx[:, :, :1]
    gidx = jnp.where(gidx == N, jnp.broadcast_to(first, gidx.shape), gidx)
    flat = gidx.reshape(B, -1)
    g = jnp.take_along_axis(table, flat[:, :, None], axis=1)
    return g.reshape(B * S * k, CT)


def _mlp_body(g_ref, c_ref, w1_ref, b1_ref, w2_ref, b2_ref, w3_ref, b3_ref,
              out_ref, *, k):
    g = g_ref[...]
    h = jnp.dot(g, w1_ref[...], preferred_element_type=jnp.float32)
    c1 = w1_ref.shape[1]
    bias1 = b1_ref[...] - jnp.dot(c_ref[...], w1_ref[32:35, :],
                                  preferred_element_type=jnp.float32)
    h = h.reshape(S_BLK, k, c1)
    h = jnp.maximum(h + bias1[:, None, :], 0.0)
    h = h.reshape(S_BLK * k, c1)
    h = jnp.maximum(jnp.dot(h, w2_ref[...],
                            preferred_element_type=jnp.float32)
                    + b2_ref[...], 0.0)
    h = jnp.maximum(jnp.dot(h, w3_ref[...],
                            preferred_element_type=jnp.float32)
                    + b3_ref[...], 0.0)
    c3 = w3_ref.shape[1]
    out_ref[...] = jnp.max(h.reshape(S_BLK, k, c3), axis=1)


def _mlp_max(g, centers, ws, bs, k):
    c1, c2, c3 = ws[0].shape[1], ws[1].shape[1], ws[2].shape[1]
    nblk = (B * S) // S_BLK
    return pl.pallas_call(
        functools.partial(_mlp_body, k=k),
        grid=(nblk,),
        in_specs=[
            pl.BlockSpec((S_BLK * k, CT), lambda i: (i, 0)),
            pl.BlockSpec((S_BLK, 3), lambda i: (i, 0)),
            pl.BlockSpec((CT, c1), lambda i: (0, 0)),
            pl.BlockSpec((1, c1), lambda i: (0, 0)),
            pl.BlockSpec((c1, c2), lambda i: (0, 0)),
            pl.BlockSpec((1, c2), lambda i: (0, 0)),
            pl.BlockSpec((c2, c3), lambda i: (0, 0)),
            pl.BlockSpec((1, c3), lambda i: (0, 0)),
        ],
        out_specs=pl.BlockSpec((S_BLK, c3), lambda i: (i, 0)),
        out_shape=jax.ShapeDtypeStruct((B * S, c3), jnp.float32),
    )(g, centers, ws[0], bs[0], ws[1], bs[1], ws[2], bs[2])


def kernel(xyz, points, W_0_0, b_0_0, W_0_1, b_0_1, W_0_2, b_0_2,
           W_1_0, b_1_0, W_1_1, b_1_1, W_1_2, b_1_2):
    weights = ((W_0_0, b_0_0, W_0_1, b_0_1, W_0_2, b_0_2),
               (W_1_0, b_1_0, W_1_1, b_1_1, W_1_2, b_1_2))

    xyz_t = jnp.transpose(xyz, (0, 2, 1))
    points_t = jnp.transpose(points, (0, 2, 1))
    table = jnp.concatenate(
        [points_t, xyz_t, jnp.zeros((B, N, CT - CIN - 3), jnp.float32)],
        axis=-1)

    new_xyz = _fps(xyz)
    centers = new_xyz.reshape(B * S, 3)

    outs = []
    for bi in range(2):
        k = KS[bi]
        w0, bb0, w1, bb1, w2, bb2 = weights[bi]
        ws = (jnp.pad(w0.T, ((0, CT - w0.shape[1]), (0, 0))), w1.T, w2.T)
        bs = (bb0[None, :], bb1[None, :], bb2[None, :])
        g = _ball_query_gather(xyz_t, table, new_xyz, RAD[bi], k)
        o = _mlp_max(g, centers, ws, bs, k)
        outs.append(o.reshape(B, S, -1))

    new_xyz_out = jnp.transpose(new_xyz, (0, 2, 1))
    new_points = jnp.transpose(jnp.concatenate(outs, axis=-1), (0, 2, 1))
    return (new_xyz_out, new_points)


# speedup vs baseline: 13.6831x; 7.9256x over previous
import functools
import jax
import jax.numpy as jnp
from jax.experimental import pallas as pl
from jax.experimental.pallas import tpu as pltpu
from jax.experimental.pallas import tpu_sc as plsc

B = 4
N = 8192
S = 512
CIN = 32
CT = 128
RAD = (0.2, 0.4)
KS = (32, 64)
S_BLK = 128


def _fps_body(xyz_ref, out_ref):
    x = xyz_ref[:, 0, :]
    y = xyz_ref[:, 1, :]
    z = xyz_ref[:, 2, :]
    lane = jax.lax.broadcasted_iota(jnp.int32, (B, N), 1)

    def step(t, carry):
        distance, farthest = carry
        sel = (lane == farthest).astype(jnp.float32)
        cx = jnp.sum(x * sel, axis=-1, keepdims=True)
        cy = jnp.sum(y * sel, axis=-1, keepdims=True)
        cz = jnp.sum(z * sel, axis=-1, keepdims=True)
        cvec = jnp.concatenate([cx, cy, cz], axis=-1)
        out_ref[:, pl.ds(t, 1), :] = cvec[:, None, :]
        dx = x - cx
        dy = y - cy
        dz = z - cz
        dist = dx * dx + dy * dy + dz * dz
        distance = jnp.minimum(distance, dist)
        m = jnp.max(distance, axis=-1, keepdims=True)
        new_far = jnp.min(jnp.where(distance == m, lane, N), axis=-1,
                          keepdims=True)
        return distance, new_far

    dist0 = jnp.full((B, N), 1e10, dtype=jnp.float32)
    far0 = jnp.zeros((B, 1), dtype=jnp.int32)
    jax.lax.fori_loop(0, S, step, (dist0, far0))


def _fps(xyz):
    return pl.pallas_call(
        _fps_body,
        out_shape=jax.ShapeDtypeStruct((B, S, 3), jnp.float32),
    )(xyz)


DBLK = 256


def _dist_body(c_ref, pts_ref, out_ref):
    c = c_ref[...]
    p = pts_ref[...][0]
    dot = jnp.dot(c, p)
    s2 = jnp.sum(c * c, axis=1, keepdims=True)
    n2 = jnp.sum(p * p, axis=0, keepdims=True)
    out_ref[...] = (s2 + n2) - 2.0 * dot


def _dist(centers, xyz):
    nj = S // DBLK
    return pl.pallas_call(
        _dist_body,
        grid=(B, nj),
        in_specs=[
            pl.BlockSpec((DBLK, 3), lambda b, j: (b * (S // DBLK) + j, 0)),
            pl.BlockSpec((1, 3, N), lambda b, j: (b, 0, 0)),
        ],
        out_specs=pl.BlockSpec((DBLK, N), lambda b, j: (b * (S // DBLK) + j, 0)),
        out_shape=jax.ShapeDtypeStruct((B * S, N), jnp.float32),
    )(centers, xyz)


NW = 32
RPW = (B * S) // NW
K0, K1 = KS
R0SQ, R1SQ = RAD[0] ** 2, RAD[1] ** 2
NCHUNK = N // 16


def _bq_body(d2_hbm, table_hbm, g0_hbm, g1_hbm,
             dbuf, sm0, sm1, cnt0, cnt1, tb0, ivb0, ivb1,
             rows0, rows1, sem):
    wid = jax.lax.axis_index("s") * 2 + jax.lax.axis_index("c")
    b = wid // (NW // B)
    row0 = wid * RPW
    boff = b * N
    lane = jax.lax.broadcasted_iota(jnp.int32, (16,), 0)

    def per_s(sl, _):
        pltpu.sync_copy(d2_hbm.at[row0 + sl], dbuf)
        cnt0[0] = 0
        cnt1[0] = 0

        def scan_chunk(j, _2):
            c0 = cnt0[0]
            c1 = cnt1[0]

            @pl.when((c0 < K0) | (c1 < K1))
            def _():
                base = j * 16
                d2 = dbuf[pl.ds(base, 16)]
                tb0[pl.ds(0, 16)] = jnp.where(d2 <= R0SQ, 1, 0)
                tb0[pl.ds(16, 16)] = jnp.where(d2 <= R1SQ, 1, 0)
                h0v = tb0[pl.ds(0, 16)]
                h1v = tb0[pl.ds(16, 16)]
                gi = boff + base
                c0i = c0
                c1i = c1
                for i in range(16):
                    h0 = h0v[i]
                    h1 = h1v[i]
                    a0 = jnp.where(h0 > 0, jnp.minimum(c0i, K0), K0)
                    a1 = jnp.where(h1 > 0, jnp.minimum(c1i, K1), K1)
                    sm0[a0] = gi + i
                    sm1[a1] = gi + i
                    c0i = c0i + h0
                    c1i = c1i + h1
                cnt0[0] = c0i
                cnt1[0] = c1i

            return _2

        jax.lax.fori_loop(0, NCHUNK, scan_chunk, 0)
        c0 = cnt0[0]
        c1 = cnt1[0]
        f0 = sm0[0]
        f1 = sm1[0]

        for w in range(K0 // 16):
            vv = jnp.full((16,), jnp.where(w * 16 < c0, sm0[w * 16], f0),
                          jnp.int32)
            for i in range(1, 16):
                si = jnp.where(w * 16 + i < c0, sm0[w * 16 + i], f0)
                vv = jnp.where(lane == i, si, vv)
            ivb0[pl.ds(w * 16, 16)] = vv
        for w in range(K1 // 16):
            vv = jnp.full((16,), jnp.where(w * 16 < c1, sm1[w * 16], f1),
                          jnp.int32)
            for i in range(1, 16):
                si = jnp.where(w * 16 + i < c1, sm1[w * 16 + i], f1)
                vv = jnp.where(lane == i, si, vv)
            ivb1[pl.ds(w * 16, 16)] = vv

        pltpu.async_copy(table_hbm.at[ivb0], rows0, sem).wait()
        pltpu.async_copy(table_hbm.at[ivb1], rows1, sem).wait()
        pltpu.sync_copy(rows0, g0_hbm.at[pl.ds((row0 + sl) * K0, K0), :])
        pltpu.sync_copy(rows1, g1_hbm.at[pl.ds((row0 + sl) * K1, K1), :])
        return 0

    jax.lax.fori_loop(0, RPW, per_s, 0)


def _ball_query_gather_sc(d2, table2d):
    mesh = plsc.VectorSubcoreMesh(core_axis_name="c", subcore_axis_name="s",
                                  num_cores=2, num_subcores=16)
    fn = pl.kernel(
        _bq_body,
        out_type=[
            jax.ShapeDtypeStruct((B * S * K0, CT), jnp.float32),
            jax.ShapeDtypeStruct((B * S * K1, CT), jnp.float32),
        ],
        mesh=mesh,
        scratch_types=[
            pltpu.VMEM((N,), jnp.float32),
            pltpu.SMEM((K0 + 1,), jnp.int32),
            pltpu.SMEM((K1 + 1,), jnp.int32),
            pltpu.SMEM((1,), jnp.int32),
            pltpu.SMEM((1,), jnp.int32),
            pltpu.VMEM((32,), jnp.int32),
            pltpu.VMEM((K0,), jnp.int32),
            pltpu.VMEM((K1,), jnp.int32),
            pltpu.VMEM((K0, CT), jnp.float32),
            pltpu.VMEM((K1, CT), jnp.float32),
            pltpu.SemaphoreType.DMA,
        ],
    )
    return fn(d2, table2d)


def _mlp_body(g_ref, c_ref, w1_ref, b1_ref, w2_ref, b2_ref, w3_ref, b3_ref,
              out_ref, *, k):
    g = g_ref[...]
    h = jnp.dot(g, w1_ref[...], preferred_element_type=jnp.float32)
    c1 = w1_ref.shape[1]
    bias1 = b1_ref[...] - jnp.dot(c_ref[...], w1_ref[32:35, :],
                                  preferred_element_type=jnp.float32)
    h = h.reshape(S_BLK, k, c1)
    h = jnp.maximum(h + bias1[:, None, :], 0.0)
    h = h.reshape(S_BLK * k, c1)
    h = jnp.maximum(jnp.dot(h, w2_ref[...],
                            preferred_element_type=jnp.float32)
                    + b2_ref[...], 0.0)
    h = jnp.maximum(jnp.dot(h, w3_ref[...],
                            preferred_element_type=jnp.float32)
                    + b3_ref[...], 0.0)
    c3 = w3_ref.shape[1]
    out_ref[...] = jnp.max(h.reshape(S_BLK, k, c3), axis=1)


def _mlp_max(g, centers, ws, bs, k):
    c1, c2, c3 = ws[0].shape[1], ws[1].shape[1], ws[2].shape[1]
    nblk = (B * S) // S_BLK
    return pl.pallas_call(
        functools.partial(_mlp_body, k=k),
        grid=(nblk,),
        in_specs=[
            pl.BlockSpec((S_BLK * k, CT), lambda i: (i, 0)),
            pl.BlockSpec((S_BLK, 3), lambda i: (i, 0)),
            pl.BlockSpec((CT, c1), lambda i: (0, 0)),
            pl.BlockSpec((1, c1), lambda i: (0, 0)),
            pl.BlockSpec((c1, c2), lambda i: (0, 0)),
            pl.BlockSpec((1, c2), lambda i: (0, 0)),
            pl.BlockSpec((c2, c3), lambda i: (0, 0)),
            pl.BlockSpec((1, c3), lambda i: (0, 0)),
        ],
        out_specs=pl.BlockSpec((S_BLK, c3), lambda i: (i, 0)),
        out_shape=jax.ShapeDtypeStruct((B * S, c3), jnp.float32),
    )(g, centers, ws[0], bs[0], ws[1], bs[1], ws[2], bs[2])


def kernel(xyz, points, W_0_0, b_0_0, W_0_1, b_0_1, W_0_2, b_0_2,
           W_1_0, b_1_0, W_1_1, b_1_1, W_1_2, b_1_2):
    weights = ((W_0_0, b_0_0, W_0_1, b_0_1, W_0_2, b_0_2),
               (W_1_0, b_1_0, W_1_1, b_1_1, W_1_2, b_1_2))

    xyz_t = jnp.transpose(xyz, (0, 2, 1))
    points_t = jnp.transpose(points, (0, 2, 1))
    table = jnp.concatenate(
        [points_t, xyz_t, jnp.zeros((B, N, CT - CIN - 3), jnp.float32)],
        axis=-1)

    new_xyz = _fps(xyz)
    centers = new_xyz.reshape(B * S, 3)

    d2 = _dist(centers, xyz)
    gs = _ball_query_gather_sc(d2, table.reshape(B * N, CT))

    outs = []
    for bi in range(2):
        k = KS[bi]
        w0, bb0, w1, bb1, w2, bb2 = weights[bi]
        ws = (jnp.pad(w0.T, ((0, CT - w0.shape[1]), (0, 0))), w1.T, w2.T)
        bs = (bb0[None, :], bb1[None, :], bb2[None, :])
        o = _mlp_max(gs[bi], centers, ws, bs, k)
        outs.append(o.reshape(B, S, -1))

    new_xyz_out = jnp.transpose(new_xyz, (0, 2, 1))
    new_points = jnp.transpose(jnp.concatenate(outs, axis=-1), (0, 2, 1))
    return (new_xyz_out, new_points)
